# Initial kernel scaffold; baseline (speedup 1.0000x reference)
import jax
import jax.numpy as jnp
from jax.experimental import pallas as pl


def kernel(world_pos, prev_world_pos, mesh_pos, node_type, cells, params):
    raise NotImplementedError("write your pallas kernel here")


# speedup vs baseline: 2.0712x; 2.0712x over previous
import functools

import jax
import jax.numpy as jnp
import numpy as np
from jax import lax
from jax.experimental import pallas as pl
from jax.experimental.pallas import tpu as pltpu
from jax.experimental.pallas import tpu_sc as plsc

N_NODES = 10000
N_CELLS = 10000
NODE_TYPE_SIZE = 9
LATENT = 128
MP_STEPS = 15

NW = 32
CHUNK = 128
E_PAD = 61440
N_PAD = 10240
ROWS_PER_W = E_PAD // (NW * CHUNK)
NODE_ROWS_PER_SUB = N_PAD // 16


def _sc_mesh():
    return plsc.VectorSubcoreMesh(core_axis_name="c", subcore_axis_name="s")


def _gather2(tbl_a, tbl_b, idx_a3d, idx_b3d):
    d = tbl_a.shape[1]

    @functools.partial(
        pl.kernel,
        out_type=(jax.ShapeDtypeStruct((E_PAD, d), jnp.float32),
                  jax.ShapeDtypeStruct((E_PAD, d), jnp.float32)),
        mesh=_sc_mesh(),
        scratch_types=[
            pltpu.VMEM((ROWS_PER_W, CHUNK), jnp.int32),
            pltpu.VMEM((ROWS_PER_W, CHUNK), jnp.int32),
            pltpu.VMEM((2 * CHUNK, d), jnp.float32),
            pltpu.VMEM((2 * CHUNK, d), jnp.float32),
        ] + [pltpu.SemaphoreType.DMA] * 8,
    )
    def k(ta, tb, ia, ib, oa, ob, iav, ibv, bufa, bufb,
          sga0, sga1, sgb0, sgb1, swa0, swa1, swb0, swb1):
        cid = lax.axis_index("c")
        sid = lax.axis_index("s")
        wid = sid * 2 + cid
        base_row = wid * ROWS_PER_W
        pltpu.sync_copy(ia.at[wid], iav)
        pltpu.sync_copy(ib.at[wid], ibv)
        sga = (sga0, sga1)
        sgb = (sgb0, sgb1)
        swa = (swa0, swa1)
        swb = (swb0, swb1)
        wh = [None, None]
        for j in range(ROWS_PER_W):
            p = j & 1
            if wh[p] is not None:
                wh[p][0].wait()
                wh[p][1].wait()
            ba = bufa.at[pl.ds(p * CHUNK, CHUNK)]
            bb = bufb.at[pl.ds(p * CHUNK, CHUNK)]
            ha = pltpu.async_copy(ta.at[iav.at[j]], ba, sga[p])
            hb = pltpu.async_copy(tb.at[ibv.at[j]], bb, sgb[p])
            ha.wait()
            hb.wait()
            r0 = (base_row + j) * CHUNK
            wh[p] = (pltpu.async_copy(ba, oa.at[pl.ds(r0, CHUNK)], swa[p]),
                     pltpu.async_copy(bb, ob.at[pl.ds(r0, CHUNK)], swb[p]))
        for p in (0, 1):
            if wh[p] is not None:
                wh[p][0].wait()
                wh[p][1].wait()

    return k(tbl_a, tbl_b, idx_a3d, idx_b3d)


def _scatter_add(edges, idx3d, zeros_tbl):
    @functools.partial(
        pl.kernel,
        out_type=jax.ShapeDtypeStruct((2 * N_PAD, LATENT), jnp.float32),
        mesh=_sc_mesh(),
        scratch_types=[
            pltpu.VMEM((ROWS_PER_W, CHUNK), jnp.int32),
            pltpu.VMEM((2 * CHUNK, LATENT), jnp.float32),
            pltpu.VMEM_SHARED((N_PAD, LATENT), jnp.float32),
        ] + [pltpu.SemaphoreType.DMA] * 2,
    )
    def k(eh, ih, zh, out, iv, ebuf, acc, sl0, sl1):
        cid = lax.axis_index("c")
        sid = lax.axis_index("s")
        wid = sid * 2 + cid
        base_row = wid * ROWS_PER_W
        nbase = sid * NODE_ROWS_PER_SUB
        pltpu.sync_copy(zh.at[pl.ds(0, CHUNK)], ebuf.at[pl.ds(0, CHUNK)])
        for t in range(NODE_ROWS_PER_SUB // CHUNK):
            pltpu.sync_copy(ebuf.at[pl.ds(0, CHUNK)],
                            acc.at[pl.ds(nbase + t * CHUNK, CHUNK)])
        pltpu.sync_copy(ih.at[wid], iv)
        plsc.subcore_barrier()
        sl = (sl0, sl1)
        hl = [None, None]
        hl[0] = pltpu.async_copy(eh.at[pl.ds(base_row * CHUNK, CHUNK)],
                                 ebuf.at[pl.ds(0, CHUNK)], sl[0])
        for j in range(ROWS_PER_W):
            p = j & 1
            if j + 1 < ROWS_PER_W:
                q = (j + 1) & 1
                r0 = (base_row + j + 1) * CHUNK
                hl[q] = pltpu.async_copy(eh.at[pl.ds(r0, CHUNK)],
                                         ebuf.at[pl.ds(q * CHUNK, CHUNK)], sl[q])
            hl[p].wait()
            pltpu.sync_copy(ebuf.at[pl.ds(p * CHUNK, CHUNK)],
                            acc.at[iv.at[j]], add=True)
        plsc.subcore_barrier()
        for t in range(NODE_ROWS_PER_SUB // CHUNK):
            p = t & 1
            r0 = nbase + t * CHUNK
            pltpu.sync_copy(acc.at[pl.ds(r0, CHUNK)],
                            ebuf.at[pl.ds(p * CHUNK, CHUNK)])
            pltpu.sync_copy(ebuf.at[pl.ds(p * CHUNK, CHUNK)],
                            out.at[pl.ds(cid * N_PAD + r0, CHUNK)])

    return k(edges, idx3d, zeros_tbl)


_F32 = jnp.float32


def _dot(a, b):
    return jnp.dot(a, b, preferred_element_type=_F32)


def _ln(h, lns, lno):
    mu = jnp.mean(h, axis=-1, keepdims=True)
    d = h - mu
    var = jnp.mean(d * d, axis=-1, keepdims=True)
    return d * lax.rsqrt(var + 1e-5) * lns + lno


def _full_spec(shape):
    return pl.BlockSpec(shape, lambda i: (0,) * len(shape))


def _row_spec(blk, width):
    return pl.BlockSpec((blk, width), lambda i: (i, 0))


def _edge_encoder_call(gsf, grf, wrel, wn, b1, w2, b2, w3, b3, lns, lno, blk):
    def body(gs_ref, gr_ref, wrel_ref, wn_ref, b1_ref, w2_ref, b2_ref,
             w3_ref, b3_ref, lns_ref, lno_ref, out_ref):
        rel = gs_ref[...] - gr_ref[...]
        s = rel * rel
        lane = lax.broadcasted_iota(jnp.int32, s.shape, 1)
        nwp = jnp.sqrt(jnp.sum(jnp.where(lane < 3, s, 0.0), -1, keepdims=True))
        nmp = jnp.sqrt(jnp.sum(jnp.where((lane >= 3) & (lane < 5), s, 0.0),
                               -1, keepdims=True))
        h = (_dot(rel, wrel_ref[...]) + nwp * wn_ref[0:1, :]
             + nmp * wn_ref[1:2, :] + b1_ref[...])
        h = jnp.maximum(h, 0.0)
        h = jnp.maximum(_dot(h, w2_ref[...]) + b2_ref[...], 0.0)
        h = _dot(h, w3_ref[...]) + b3_ref[...]
        out_ref[...] = _ln(h, lns_ref[...], lno_ref[...])

    grid = (E_PAD // blk,)
    return pl.pallas_call(
        body,
        grid=grid,
        in_specs=[_row_spec(blk, LATENT), _row_spec(blk, LATENT),
                  _full_spec((LATENT, LATENT)), _full_spec((8, LATENT)),
                  _full_spec((1, LATENT)), _full_spec((LATENT, LATENT)),
                  _full_spec((1, LATENT)), _full_spec((LATENT, LATENT)),
                  _full_spec((1, LATENT)), _full_spec((1, LATENT)),
                  _full_spec((1, LATENT))],
        out_specs=_row_spec(blk, LATENT),
        out_shape=jax.ShapeDtypeStruct((E_PAD, LATENT), _F32),
    )(gsf, grf, wrel, wn, b1, w2, b2, w3, b3, lns, lno)


def _node_encoder_call(nf, w1, b1, w2, b2, w3, b3, lns, lno, ws, wr, blk):
    def body(nf_ref, w1_ref, b1_ref, w2_ref, b2_ref, w3_ref, b3_ref,
             lns_ref, lno_ref, ws_ref, wr_ref, out_ref, ns_ref, nr_ref):
        h = _dot(nf_ref[...], w1_ref[...]) + b1_ref[...]
        h = jnp.maximum(h, 0.0)
        h = jnp.maximum(_dot(h, w2_ref[...]) + b2_ref[...], 0.0)
        h = _dot(h, w3_ref[...]) + b3_ref[...]
        out = _ln(h, lns_ref[...], lno_ref[...])
        out_ref[...] = out
        ns_ref[...] = _dot(out, ws_ref[...])
        nr_ref[...] = _dot(out, wr_ref[...])

    grid = (N_PAD // blk,)
    sds = jax.ShapeDtypeStruct((N_PAD, LATENT), _F32)
    return pl.pallas_call(
        body,
        grid=grid,
        in_specs=[_row_spec(blk, 16), _full_spec((16, LATENT)),
                  _full_spec((1, LATENT)), _full_spec((LATENT, LATENT)),
                  _full_spec((1, LATENT)), _full_spec((LATENT, LATENT)),
                  _full_spec((1, LATENT)), _full_spec((1, LATENT)),
                  _full_spec((1, LATENT)), _full_spec((LATENT, LATENT)),
                  _full_spec((LATENT, LATENT))],
        out_specs=[_row_spec(blk, LATENT)] * 3,
        out_shape=[sds, sds, sds],
    )(nf, w1, b1, w2, b2, w3, b3, lns, lno, ws, wr)


def _edge_step_call(el, gs, gr, w1a, b1, w2, b2, w3, b3, lns, lno, blk):
    def body(el_ref, gs_ref, gr_ref, w1_ref, b1_ref, w2_ref, b2_ref,
             w3_ref, b3_ref, lns_ref, lno_ref, out_ref):
        x = el_ref[...]
        h = _dot(x, w1_ref[...]) + gs_ref[...] + gr_ref[...] + b1_ref[...]
        h = jnp.maximum(h, 0.0)
        h = jnp.maximum(_dot(h, w2_ref[...]) + b2_ref[...], 0.0)
        h = _dot(h, w3_ref[...]) + b3_ref[...]
        out_ref[...] = x + _ln(h, lns_ref[...], lno_ref[...])

    grid = (E_PAD // blk,)
    return pl.pallas_call(
        body,
        grid=grid,
        in_specs=[_row_spec(blk, LATENT)] * 3 + [
            _full_spec((LATENT, LATENT)), _full_spec((1, LATENT)),
            _full_spec((LATENT, LATENT)), _full_spec((1, LATENT)),
            _full_spec((LATENT, LATENT)), _full_spec((1, LATENT)),
            _full_spec((1, LATENT)), _full_spec((1, LATENT))],
        out_specs=_row_spec(blk, LATENT),
        out_shape=jax.ShapeDtypeStruct((E_PAD, LATENT), _F32),
    )(el, gs, gr, w1a, b1, w2, b2, w3, b3, lns, lno)


def _node_step_call(nl, p0, p1, wna, wnb, b1, w2, b2, w3, b3, lns, lno,
                    ws, wr, blk, want_tables):
    def body_t(nl_ref, p0_ref, p1_ref, wna_ref, wnb_ref, b1_ref, w2_ref,
               b2_ref, w3_ref, b3_ref, lns_ref, lno_ref, ws_ref, wr_ref,
               out_ref, ns_ref, nr_ref):
        x = nl_ref[...]
        agg = p0_ref[...] + p1_ref[...]
        h = _dot(x, wna_ref[...]) + _dot(agg, wnb_ref[...]) + b1_ref[...]
        h = jnp.maximum(h, 0.0)
        h = jnp.maximum(_dot(h, w2_ref[...]) + b2_ref[...], 0.0)
        h = _dot(h, w3_ref[...]) + b3_ref[...]
        out = x + _ln(h, lns_ref[...], lno_ref[...])
        out_ref[...] = out
        ns_ref[...] = _dot(out, ws_ref[...])
        nr_ref[...] = _dot(out, wr_ref[...])

    def body_p(nl_ref, p0_ref, p1_ref, wna_ref, wnb_ref, b1_ref, w2_ref,
               b2_ref, w3_ref, b3_ref, lns_ref, lno_ref, out_ref):
        x = nl_ref[...]
        agg = p0_ref[...] + p1_ref[...]
        h = _dot(x, wna_ref[...]) + _dot(agg, wnb_ref[...]) + b1_ref[...]
        h = jnp.maximum(h, 0.0)
        h = jnp.maximum(_dot(h, w2_ref[...]) + b2_ref[...], 0.0)
        h = _dot(h, w3_ref[...]) + b3_ref[...]
        out_ref[...] = x + _ln(h, lns_ref[...], lno_ref[...])

    grid = (N_PAD // blk,)
    sds = jax.ShapeDtypeStruct((N_PAD, LATENT), _F32)
    base_specs = [_row_spec(blk, LATENT)] * 3 + [
        _full_spec((LATENT, LATENT)), _full_spec((LATENT, LATENT)),
        _full_spec((1, LATENT)), _full_spec((LATENT, LATENT)),
        _full_spec((1, LATENT)), _full_spec((LATENT, LATENT)),
        _full_spec((1, LATENT)), _full_spec((1, LATENT)),
        _full_spec((1, LATENT))]
    if want_tables:
        return pl.pallas_call(
            body_t,
            grid=grid,
            in_specs=base_specs + [_full_spec((LATENT, LATENT))] * 2,
            out_specs=[_row_spec(blk, LATENT)] * 3,
            out_shape=[sds, sds, sds],
        )(nl, p0, p1, wna, wnb, b1, w2, b2, w3, b3, lns, lno, ws, wr)
    return pl.pallas_call(
        body_p,
        grid=grid,
        in_specs=base_specs,
        out_specs=_row_spec(blk, LATENT),
        out_shape=sds,
    )(nl, p0, p1, wna, wnb, b1, w2, b2, w3, b3, lns, lno)


def _decoder_call(nl, w1, b1, w2, b2, w3, b3, blk):
    def body(nl_ref, w1_ref, b1_ref, w2_ref, b2_ref, w3_ref, b3_ref, out_ref):
        h = jnp.maximum(_dot(nl_ref[...], w1_ref[...]) + b1_ref[...], 0.0)
        h = jnp.maximum(_dot(h, w2_ref[...]) + b2_ref[...], 0.0)
        out_ref[...] = _dot(h, w3_ref[...]) + b3_ref[...]

    grid = (N_PAD // blk,)
    return pl.pallas_call(
        body,
        grid=grid,
        in_specs=[_row_spec(blk, LATENT),
                  _full_spec((LATENT, LATENT)), _full_spec((1, LATENT)),
                  _full_spec((LATENT, LATENT)), _full_spec((1, LATENT)),
                  _full_spec((LATENT, LATENT)), _full_spec((1, LATENT))],
        out_specs=_row_spec(blk, LATENT),
        out_shape=jax.ShapeDtypeStruct((N_PAD, LATENT), _F32),
    )(nl, w1, b1, w2, b2, w3, b3)


def _edges(cells):
    edges = jnp.concatenate([cells[:, 0:2], cells[:, 1:3],
                             jnp.stack([cells[:, 2], cells[:, 0]], axis=-1)], axis=0)
    lo = jnp.minimum(edges[:, 0], edges[:, 1]).astype(jnp.int64)
    hi = jnp.maximum(edges[:, 0], edges[:, 1]).astype(jnp.int64)
    mask = lo != hi
    sentinel = N_NODES * N_NODES
    packed_all = jnp.where(mask, lo * N_NODES + hi, sentinel)
    n_edges = edges.shape[0]
    packed = jnp.unique(packed_all, size=n_edges, fill_value=sentinel)
    valid = packed != sentinel
    lo_u = (packed // N_NODES).astype(jnp.int32)
    hi_u = (packed % N_NODES).astype(jnp.int32)
    lo_g = jnp.where(valid, lo_u, 0)
    hi_g = jnp.where(valid, hi_u, 0)
    lo_s = jnp.where(valid, lo_u, N_NODES)
    hi_s = jnp.where(valid, hi_u, N_NODES)
    senders = jnp.concatenate([lo_g, hi_g], axis=0)
    receivers = jnp.concatenate([hi_s, lo_s], axis=0)
    return senders, receivers


def _vec(x):
    return x.reshape(1, -1)


def kernel(world_pos, prev_world_pos, mesh_pos, node_type, cells, params):
    senders, receivers = _edges(cells)
    n_e = senders.shape[0]
    s_pad = jnp.concatenate(
        [senders, jnp.zeros((E_PAD - n_e,), jnp.int32)]).astype(jnp.int32)
    r_pad = jnp.concatenate(
        [receivers, jnp.full((E_PAD - n_e,), N_NODES, jnp.int32)]).astype(jnp.int32)
    s2d = s_pad.reshape(NW, ROWS_PER_W, CHUNK)
    r2d = r_pad.reshape(NW, ROWS_PER_W, CHUNK)

    vel = world_pos - prev_world_pos
    onehot = jax.nn.one_hot(node_type[:, 0], NODE_TYPE_SIZE, dtype=_F32)
    nf = jnp.concatenate([vel, onehot], axis=-1)
    nf16 = jnp.zeros((N_PAD, 16), _F32).at[:N_NODES, :12].set(nf)

    enc_n = params["node_encoder"]
    n_std = params["node_norm_std"]
    n_mean = params["node_norm_mean"]
    w1n = enc_n["layers"][0]["w"] / n_std[:, None]
    b1n = enc_n["layers"][0]["b"] - (n_mean / n_std) @ enc_n["layers"][0]["w"]
    w1n16 = jnp.zeros((16, LATENT), _F32).at[:12, :].set(w1n)

    ftbl = (jnp.zeros((N_PAD, LATENT), _F32)
            .at[:N_NODES, 0:3].set(world_pos)
            .at[:N_NODES, 3:5].set(mesh_pos))

    enc_e = params["edge_encoder"]
    e_std = params["edge_norm_std"]
    e_mean = params["edge_norm_mean"]
    w1e = enc_e["layers"][0]["w"] / e_std[:, None]
    b1e = enc_e["layers"][0]["b"] - (e_mean / e_std) @ enc_e["layers"][0]["w"]
    wrel = (jnp.zeros((LATENT, LATENT), _F32)
            .at[0:3, :].set(w1e[0:3, :])
            .at[3:5, :].set(w1e[4:6, :]))
    wn = (jnp.zeros((8, LATENT), _F32)
          .at[0, :].set(w1e[3, :])
          .at[1, :].set(w1e[6, :]))

    blocks = params["blocks"]
    ws0 = blocks[0]["edge"]["layers"][0]["w"][LATENT:2 * LATENT, :]
    wr0 = blocks[0]["edge"]["layers"][0]["w"][2 * LATENT:, :]

    zeros_tbl = jnp.zeros((N_PAD, LATENT), _F32)

    gsf, grf = _gather2(ftbl, ftbl, s2d, r2d)
    el = _edge_encoder_call(
        gsf, grf, wrel, wn, _vec(b1e),
        enc_e["layers"][1]["w"], _vec(enc_e["layers"][1]["b"]),
        enc_e["layers"][2]["w"], _vec(enc_e["layers"][2]["b"]),
        _vec(enc_e["ln_scale"]), _vec(enc_e["ln_offset"]), blk=1024)
    nl, ns, nr = _node_encoder_call(
        nf16, w1n16, _vec(b1n),
        enc_n["layers"][1]["w"], _vec(enc_n["layers"][1]["b"]),
        enc_n["layers"][2]["w"], _vec(enc_n["layers"][2]["b"]),
        _vec(enc_n["ln_scale"]), _vec(enc_n["ln_offset"]),
        ws0, wr0, blk=1024)

    for i in range(MP_STEPS):
        blk_e = blocks[i]["edge"]
        blk_n = blocks[i]["node"]
        gs, gr = _gather2(ns, nr, s2d, r2d)
        el = _edge_step_call(
            el, gs, gr,
            blk_e["layers"][0]["w"][:LATENT, :], _vec(blk_e["layers"][0]["b"]),
            blk_e["layers"][1]["w"], _vec(blk_e["layers"][1]["b"]),
            blk_e["layers"][2]["w"], _vec(blk_e["layers"][2]["b"]),
            _vec(blk_e["ln_scale"]), _vec(blk_e["ln_offset"]), blk=1024)
        parts = _scatter_add(el, r2d, zeros_tbl)
        p0 = parts[:N_PAD]
        p1 = parts[N_PAD:]
        want_tables = i + 1 < MP_STEPS
        if want_tables:
            wsn = blocks[i + 1]["edge"]["layers"][0]["w"][LATENT:2 * LATENT, :]
            wrn = blocks[i + 1]["edge"]["layers"][0]["w"][2 * LATENT:, :]
            nl, ns, nr = _node_step_call(
                nl, p0, p1,
                blk_n["layers"][0]["w"][:LATENT, :],
                blk_n["layers"][0]["w"][LATENT:, :],
                _vec(blk_n["layers"][0]["b"]),
                blk_n["layers"][1]["w"], _vec(blk_n["layers"][1]["b"]),
                blk_n["layers"][2]["w"], _vec(blk_n["layers"][2]["b"]),
                _vec(blk_n["ln_scale"]), _vec(blk_n["ln_offset"]),
                wsn, wrn, blk=1024, want_tables=True)
        else:
            nl = _node_step_call(
                nl, p0, p1,
                blk_n["layers"][0]["w"][:LATENT, :],
                blk_n["layers"][0]["w"][LATENT:, :],
                _vec(blk_n["layers"][0]["b"]),
                blk_n["layers"][1]["w"], _vec(blk_n["layers"][1]["b"]),
                blk_n["layers"][2]["w"], _vec(blk_n["layers"][2]["b"]),
                _vec(blk_n["ln_scale"]), _vec(blk_n["ln_offset"]),
                None, None, blk=1024, want_tables=False)

    dec = params["decoder"]
    o_std = params["out_norm_std"]
    o_mean = params["out_norm_mean"]
    w3d = jnp.zeros((LATENT, LATENT), _F32).at[:, :3].set(
        dec["layers"][2]["w"] * o_std[None, :])
    b3d = jnp.zeros((LATENT,), _F32).at[:3].set(
        dec["layers"][2]["b"] * o_std + o_mean)
    acc = _decoder_call(
        nl, dec["layers"][0]["w"], _vec(dec["layers"][0]["b"]),
        dec["layers"][1]["w"], _vec(dec["layers"][1]["b"]),
        w3d, _vec(b3d), blk=1024)

    return 2 * world_pos + acc[:N_NODES, :3] - prev_world_pos


# speedup vs baseline: 2.1515x; 1.0387x over previous
import functools

import jax
import jax.numpy as jnp
import numpy as np
from jax import lax
from jax.experimental import pallas as pl
from jax.experimental.pallas import tpu as pltpu
from jax.experimental.pallas import tpu_sc as plsc

N_NODES = 10000
N_CELLS = 10000
NODE_TYPE_SIZE = 9
LATENT = 128
MP_STEPS = 15

NW = 32
CHUNK = 128
E_PAD = 61440
N_PAD = 10240
ROWS_PER_W = E_PAD // (NW * CHUNK)
NODE_ROWS_PER_SUB = N_PAD // 16


def _sc_mesh():
    return plsc.VectorSubcoreMesh(core_axis_name="c", subcore_axis_name="s")


def _gather2(tbl_a, tbl_b, idx_a3d, idx_b3d):
    d = tbl_a.shape[1]

    nbuf = 3

    @functools.partial(
        pl.kernel,
        out_type=(jax.ShapeDtypeStruct((E_PAD, d), jnp.float32),
                  jax.ShapeDtypeStruct((E_PAD, d), jnp.float32)),
        mesh=_sc_mesh(),
        scratch_types=[
            pltpu.VMEM((ROWS_PER_W, CHUNK), jnp.int32),
            pltpu.VMEM((ROWS_PER_W, CHUNK), jnp.int32),
            pltpu.VMEM((nbuf * CHUNK, d), jnp.float32),
            pltpu.VMEM((nbuf * CHUNK, d), jnp.float32),
        ] + [pltpu.SemaphoreType.DMA] * (4 * nbuf),
    )
    def k(ta, tb, ia, ib, oa, ob, iav, ibv, bufa, bufb, *sems):
        cid = lax.axis_index("c")
        sid = lax.axis_index("s")
        wid = sid * 2 + cid
        base_row = wid * ROWS_PER_W
        pltpu.sync_copy(ia.at[wid], iav)
        pltpu.sync_copy(ib.at[wid], ibv)
        sga = sems[0:nbuf]
        sgb = sems[nbuf:2 * nbuf]
        swa = sems[2 * nbuf:3 * nbuf]
        swb = sems[3 * nbuf:4 * nbuf]
        gh = [None] * nbuf
        wh = [None] * nbuf

        def issue_gather(j):
            p = j % nbuf
            ba = bufa.at[pl.ds(p * CHUNK, CHUNK)]
            bb = bufb.at[pl.ds(p * CHUNK, CHUNK)]
            gh[p] = (pltpu.async_copy(ta.at[iav.at[j]], ba, sga[p]),
                     pltpu.async_copy(tb.at[ibv.at[j]], bb, sgb[p]))

        for j in range(min(2, ROWS_PER_W)):
            issue_gather(j)
        for j in range(ROWS_PER_W):
            p = j % nbuf
            jn = j + 2
            if jn < ROWS_PER_W:
                q = jn % nbuf
                if wh[q] is not None:
                    wh[q][0].wait()
                    wh[q][1].wait()
                    wh[q] = None
                issue_gather(jn)
            gh[p][0].wait()
            gh[p][1].wait()
            r0 = (base_row + j) * CHUNK
            ba = bufa.at[pl.ds(p * CHUNK, CHUNK)]
            bb = bufb.at[pl.ds(p * CHUNK, CHUNK)]
            wh[p] = (pltpu.async_copy(ba, oa.at[pl.ds(r0, CHUNK)], swa[p]),
                     pltpu.async_copy(bb, ob.at[pl.ds(r0, CHUNK)], swb[p]))
        for p in range(nbuf):
            if wh[p] is not None:
                wh[p][0].wait()
                wh[p][1].wait()

    return k(tbl_a, tbl_b, idx_a3d, idx_b3d)


def _scatter_add(edges, idx3d, zeros_tbl):
    @functools.partial(
        pl.kernel,
        out_type=jax.ShapeDtypeStruct((2 * N_PAD, LATENT), jnp.float32),
        mesh=_sc_mesh(),
        scratch_types=[
            pltpu.VMEM((ROWS_PER_W, CHUNK), jnp.int32),
            pltpu.VMEM((2 * CHUNK, LATENT), jnp.float32),
            pltpu.VMEM_SHARED((N_PAD, LATENT), jnp.float32),
        ] + [pltpu.SemaphoreType.DMA] * 4,
    )
    def k(eh, ih, zh, out, iv, ebuf, acc, *sems):
        nbuf = 2
        cid = lax.axis_index("c")
        sid = lax.axis_index("s")
        wid = sid * 2 + cid
        base_row = wid * ROWS_PER_W
        nbase = sid * NODE_ROWS_PER_SUB
        pltpu.sync_copy(zh.at[pl.ds(0, CHUNK)], ebuf.at[pl.ds(0, CHUNK)])
        for t in range(NODE_ROWS_PER_SUB // CHUNK):
            pltpu.sync_copy(ebuf.at[pl.ds(0, CHUNK)],
                            acc.at[pl.ds(nbase + t * CHUNK, CHUNK)])
        pltpu.sync_copy(ih.at[wid], iv)
        plsc.subcore_barrier()
        sl = sems[0:nbuf]
        ss = sems[nbuf:2 * nbuf]
        hl = [None] * nbuf
        hs = [None] * nbuf

        def issue_load(j):
            p = j % nbuf
            r0 = (base_row + j) * CHUNK
            hl[p] = pltpu.async_copy(eh.at[pl.ds(r0, CHUNK)],
                                     ebuf.at[pl.ds(p * CHUNK, CHUNK)], sl[p])

        for j in range(min(2, ROWS_PER_W)):
            issue_load(j)
        for j in range(ROWS_PER_W):
            p = j % nbuf
            hl[p].wait()
            hs[p] = pltpu.async_copy(ebuf.at[pl.ds(p * CHUNK, CHUNK)],
                                     acc.at[iv.at[j]], ss[p], add=True)
            jn = j + 2
            if jn < ROWS_PER_W:
                hs[p].wait()
                hs[p] = None
                issue_load(jn)
        for p in range(nbuf):
            if hs[p] is not None:
                hs[p].wait()
        plsc.subcore_barrier()
        for t in range(NODE_ROWS_PER_SUB // CHUNK):
            p = t & 1
            r0 = nbase + t * CHUNK
            pltpu.sync_copy(acc.at[pl.ds(r0, CHUNK)],
                            ebuf.at[pl.ds(p * CHUNK, CHUNK)])
            pltpu.sync_copy(ebuf.at[pl.ds(p * CHUNK, CHUNK)],
                            out.at[pl.ds(cid * N_PAD + r0, CHUNK)])

    return k(edges, idx3d, zeros_tbl)


_F32 = jnp.float32


def _dot(a, b):
    return jnp.dot(a, b, preferred_element_type=_F32)


def _ln(h, lns, lno):
    mu = jnp.mean(h, axis=-1, keepdims=True)
    d = h - mu
    var = jnp.mean(d * d, axis=-1, keepdims=True)
    return d * lax.rsqrt(var + 1e-5) * lns + lno


def _full_spec(shape):
    return pl.BlockSpec(shape, lambda i: (0,) * len(shape))


def _row_spec(blk, width):
    return pl.BlockSpec((blk, width), lambda i: (i, 0))


def _edge_encoder_call(gsf, grf, wrel, wn, b1, w2, b2, w3, b3, lns, lno, blk):
    def body(gs_ref, gr_ref, wrel_ref, wn_ref, b1_ref, w2_ref, b2_ref,
             w3_ref, b3_ref, lns_ref, lno_ref, out_ref):
        rel = gs_ref[...] - gr_ref[...]
        s = rel * rel
        lane = lax.broadcasted_iota(jnp.int32, s.shape, 1)
        nwp = jnp.sqrt(jnp.sum(jnp.where(lane < 3, s, 0.0), -1, keepdims=True))
        nmp = jnp.sqrt(jnp.sum(jnp.where((lane >= 3) & (lane < 5), s, 0.0),
                               -1, keepdims=True))
        h = (_dot(rel, wrel_ref[...]) + nwp * wn_ref[0:1, :]
             + nmp * wn_ref[1:2, :] + b1_ref[...])
        h = jnp.maximum(h, 0.0)
        h = jnp.maximum(_dot(h, w2_ref[...]) + b2_ref[...], 0.0)
        h = _dot(h, w3_ref[...]) + b3_ref[...]
        out_ref[...] = _ln(h, lns_ref[...], lno_ref[...])

    grid = (E_PAD // blk,)
    return pl.pallas_call(
        body,
        grid=grid,
        in_specs=[_row_spec(blk, LATENT), _row_spec(blk, LATENT),
                  _full_spec((LATENT, LATENT)), _full_spec((8, LATENT)),
                  _full_spec((1, LATENT)), _full_spec((LATENT, LATENT)),
                  _full_spec((1, LATENT)), _full_spec((LATENT, LATENT)),
                  _full_spec((1, LATENT)), _full_spec((1, LATENT)),
                  _full_spec((1, LATENT))],
        out_specs=_row_spec(blk, LATENT),
        out_shape=jax.ShapeDtypeStruct((E_PAD, LATENT), _F32),
    )(gsf, grf, wrel, wn, b1, w2, b2, w3, b3, lns, lno)


def _node_encoder_call(nf, w1, b1, w2, b2, w3, b3, lns, lno, ws, wr, blk):
    def body(nf_ref, w1_ref, b1_ref, w2_ref, b2_ref, w3_ref, b3_ref,
             lns_ref, lno_ref, ws_ref, wr_ref, out_ref, ns_ref, nr_ref):
        h = _dot(nf_ref[...], w1_ref[...]) + b1_ref[...]
        h = jnp.maximum(h, 0.0)
        h = jnp.maximum(_dot(h, w2_ref[...]) + b2_ref[...], 0.0)
        h = _dot(h, w3_ref[...]) + b3_ref[...]
        out = _ln(h, lns_ref[...], lno_ref[...])
        out_ref[...] = out
        ns_ref[...] = _dot(out, ws_ref[...])
        nr_ref[...] = _dot(out, wr_ref[...])

    grid = (N_PAD // blk,)
    sds = jax.ShapeDtypeStruct((N_PAD, LATENT), _F32)
    return pl.pallas_call(
        body,
        grid=grid,
        in_specs=[_row_spec(blk, 16), _full_spec((16, LATENT)),
                  _full_spec((1, LATENT)), _full_spec((LATENT, LATENT)),
                  _full_spec((1, LATENT)), _full_spec((LATENT, LATENT)),
                  _full_spec((1, LATENT)), _full_spec((1, LATENT)),
                  _full_spec((1, LATENT)), _full_spec((LATENT, LATENT)),
                  _full_spec((LATENT, LATENT))],
        out_specs=[_row_spec(blk, LATENT)] * 3,
        out_shape=[sds, sds, sds],
    )(nf, w1, b1, w2, b2, w3, b3, lns, lno, ws, wr)


def _edge_step_call(el, gs, gr, w1a, b1, w2, b2, w3, b3, lns, lno, blk):
    def body(el_ref, gs_ref, gr_ref, w1_ref, b1_ref, w2_ref, b2_ref,
             w3_ref, b3_ref, lns_ref, lno_ref, out_ref):
        x = el_ref[...]
        h = _dot(x, w1_ref[...]) + gs_ref[...] + gr_ref[...] + b1_ref[...]
        h = jnp.maximum(h, 0.0)
        h = jnp.maximum(_dot(h, w2_ref[...]) + b2_ref[...], 0.0)
        h = _dot(h, w3_ref[...]) + b3_ref[...]
        out_ref[...] = x + _ln(h, lns_ref[...], lno_ref[...])

    grid = (E_PAD // blk,)
    return pl.pallas_call(
        body,
        grid=grid,
        in_specs=[_row_spec(blk, LATENT)] * 3 + [
            _full_spec((LATENT, LATENT)), _full_spec((1, LATENT)),
            _full_spec((LATENT, LATENT)), _full_spec((1, LATENT)),
            _full_spec((LATENT, LATENT)), _full_spec((1, LATENT)),
            _full_spec((1, LATENT)), _full_spec((1, LATENT))],
        out_specs=_row_spec(blk, LATENT),
        out_shape=jax.ShapeDtypeStruct((E_PAD, LATENT), _F32),
    )(el, gs, gr, w1a, b1, w2, b2, w3, b3, lns, lno)


def _node_step_call(nl, p0, p1, wna, wnb, b1, w2, b2, w3, b3, lns, lno,
                    ws, wr, blk, want_tables):
    def body_t(nl_ref, p0_ref, p1_ref, wna_ref, wnb_ref, b1_ref, w2_ref,
               b2_ref, w3_ref, b3_ref, lns_ref, lno_ref, ws_ref, wr_ref,
               out_ref, ns_ref, nr_ref):
        x = nl_ref[...]
        agg = p0_ref[...] + p1_ref[...]
        h = _dot(x, wna_ref[...]) + _dot(agg, wnb_ref[...]) + b1_ref[...]
        h = jnp.maximum(h, 0.0)
        h = jnp.maximum(_dot(h, w2_ref[...]) + b2_ref[...], 0.0)
        h = _dot(h, w3_ref[...]) + b3_ref[...]
        out = x + _ln(h, lns_ref[...], lno_ref[...])
        out_ref[...] = out
        ns_ref[...] = _dot(out, ws_ref[...])
        nr_ref[...] = _dot(out, wr_ref[...])

    def body_p(nl_ref, p0_ref, p1_ref, wna_ref, wnb_ref, b1_ref, w2_ref,
               b2_ref, w3_ref, b3_ref, lns_ref, lno_ref, out_ref):
        x = nl_ref[...]
        agg = p0_ref[...] + p1_ref[...]
        h = _dot(x, wna_ref[...]) + _dot(agg, wnb_ref[...]) + b1_ref[...]
        h = jnp.maximum(h, 0.0)
        h = jnp.maximum(_dot(h, w2_ref[...]) + b2_ref[...], 0.0)
        h = _dot(h, w3_ref[...]) + b3_ref[...]
        out_ref[...] = x + _ln(h, lns_ref[...], lno_ref[...])

    grid = (N_PAD // blk,)
    sds = jax.ShapeDtypeStruct((N_PAD, LATENT), _F32)
    base_specs = [_row_spec(blk, LATENT)] * 3 + [
        _full_spec((LATENT, LATENT)), _full_spec((LATENT, LATENT)),
        _full_spec((1, LATENT)), _full_spec((LATENT, LATENT)),
        _full_spec((1, LATENT)), _full_spec((LATENT, LATENT)),
        _full_spec((1, LATENT)), _full_spec((1, LATENT)),
        _full_spec((1, LATENT))]
    if want_tables:
        return pl.pallas_call(
            body_t,
            grid=grid,
            in_specs=base_specs + [_full_spec((LATENT, LATENT))] * 2,
            out_specs=[_row_spec(blk, LATENT)] * 3,
            out_shape=[sds, sds, sds],
        )(nl, p0, p1, wna, wnb, b1, w2, b2, w3, b3, lns, lno, ws, wr)
    return pl.pallas_call(
        body_p,
        grid=grid,
        in_specs=base_specs,
        out_specs=_row_spec(blk, LATENT),
        out_shape=sds,
    )(nl, p0, p1, wna, wnb, b1, w2, b2, w3, b3, lns, lno)


def _decoder_call(nl, w1, b1, w2, b2, w3, b3, blk):
    def body(nl_ref, w1_ref, b1_ref, w2_ref, b2_ref, w3_ref, b3_ref, out_ref):
        h = jnp.maximum(_dot(nl_ref[...], w1_ref[...]) + b1_ref[...], 0.0)
        h = jnp.maximum(_dot(h, w2_ref[...]) + b2_ref[...], 0.0)
        out_ref[...] = _dot(h, w3_ref[...]) + b3_ref[...]

    grid = (N_PAD // blk,)
    return pl.pallas_call(
        body,
        grid=grid,
        in_specs=[_row_spec(blk, LATENT),
                  _full_spec((LATENT, LATENT)), _full_spec((1, LATENT)),
                  _full_spec((LATENT, LATENT)), _full_spec((1, LATENT)),
                  _full_spec((LATENT, LATENT)), _full_spec((1, LATENT))],
        out_specs=_row_spec(blk, LATENT),
        out_shape=jax.ShapeDtypeStruct((N_PAD, LATENT), _F32),
    )(nl, w1, b1, w2, b2, w3, b3)


def _edges(cells):
    edges = jnp.concatenate([cells[:, 0:2], cells[:, 1:3],
                             jnp.stack([cells[:, 2], cells[:, 0]], axis=-1)], axis=0)
    lo = jnp.minimum(edges[:, 0], edges[:, 1]).astype(jnp.int64)
    hi = jnp.maximum(edges[:, 0], edges[:, 1]).astype(jnp.int64)
    mask = lo != hi
    sentinel = N_NODES * N_NODES
    packed_all = jnp.where(mask, lo * N_NODES + hi, sentinel)
    n_edges = edges.shape[0]
    packed = jnp.unique(packed_all, size=n_edges, fill_value=sentinel)
    valid = packed != sentinel
    lo_u = (packed // N_NODES).astype(jnp.int32)
    hi_u = (packed % N_NODES).astype(jnp.int32)
    lo_g = jnp.where(valid, lo_u, 0)
    hi_g = jnp.where(valid, hi_u, 0)
    lo_s = jnp.where(valid, lo_u, N_NODES)
    hi_s = jnp.where(valid, hi_u, N_NODES)
    senders = jnp.concatenate([lo_g, hi_g], axis=0)
    receivers = jnp.concatenate([hi_s, lo_s], axis=0)
    return senders, receivers


def _vec(x):
    return x.reshape(1, -1)


def kernel(world_pos, prev_world_pos, mesh_pos, node_type, cells, params):
    senders, receivers = _edges(cells)
    n_e = senders.shape[0]
    s_pad = jnp.concatenate(
        [senders, jnp.zeros((E_PAD - n_e,), jnp.int32)]).astype(jnp.int32)
    r_pad = jnp.concatenate(
        [receivers, jnp.full((E_PAD - n_e,), N_NODES, jnp.int32)]).astype(jnp.int32)
    s2d = s_pad.reshape(NW, ROWS_PER_W, CHUNK)
    r2d = r_pad.reshape(NW, ROWS_PER_W, CHUNK)

    vel = world_pos - prev_world_pos
    onehot = jax.nn.one_hot(node_type[:, 0], NODE_TYPE_SIZE, dtype=_F32)
    nf = jnp.concatenate([vel, onehot], axis=-1)
    nf16 = jnp.zeros((N_PAD, 16), _F32).at[:N_NODES, :12].set(nf)

    enc_n = params["node_encoder"]
    n_std = params["node_norm_std"]
    n_mean = params["node_norm_mean"]
    w1n = enc_n["layers"][0]["w"] / n_std[:, None]
    b1n = enc_n["layers"][0]["b"] - (n_mean / n_std) @ enc_n["layers"][0]["w"]
    w1n16 = jnp.zeros((16, LATENT), _F32).at[:12, :].set(w1n)

    ftbl = (jnp.zeros((N_PAD, LATENT), _F32)
            .at[:N_NODES, 0:3].set(world_pos)
            .at[:N_NODES, 3:5].set(mesh_pos))

    enc_e = params["edge_encoder"]
    e_std = params["edge_norm_std"]
    e_mean = params["edge_norm_mean"]
    w1e = enc_e["layers"][0]["w"] / e_std[:, None]
    b1e = enc_e["layers"][0]["b"] - (e_mean / e_std) @ enc_e["layers"][0]["w"]
    wrel = (jnp.zeros((LATENT, LATENT), _F32)
            .at[0:3, :].set(w1e[0:3, :])
            .at[3:5, :].set(w1e[4:6, :]))
    wn = (jnp.zeros((8, LATENT), _F32)
          .at[0, :].set(w1e[3, :])
          .at[1, :].set(w1e[6, :]))

    blocks = params["blocks"]
    ws0 = blocks[0]["edge"]["layers"][0]["w"][LATENT:2 * LATENT, :]
    wr0 = blocks[0]["edge"]["layers"][0]["w"][2 * LATENT:, :]

    zeros_tbl = jnp.zeros((N_PAD, LATENT), _F32)

    gsf, grf = _gather2(ftbl, ftbl, s2d, r2d)
    el = _edge_encoder_call(
        gsf, grf, wrel, wn, _vec(b1e),
        enc_e["layers"][1]["w"], _vec(enc_e["layers"][1]["b"]),
        enc_e["layers"][2]["w"], _vec(enc_e["layers"][2]["b"]),
        _vec(enc_e["ln_scale"]), _vec(enc_e["ln_offset"]), blk=1024)
    nl, ns, nr = _node_encoder_call(
        nf16, w1n16, _vec(b1n),
        enc_n["layers"][1]["w"], _vec(enc_n["layers"][1]["b"]),
        enc_n["layers"][2]["w"], _vec(enc_n["layers"][2]["b"]),
        _vec(enc_n["ln_scale"]), _vec(enc_n["ln_offset"]),
        ws0, wr0, blk=1024)

    for i in range(MP_STEPS):
        blk_e = blocks[i]["edge"]
        blk_n = blocks[i]["node"]
        gs, gr = _gather2(ns, nr, s2d, r2d)
        el = _edge_step_call(
            el, gs, gr,
            blk_e["layers"][0]["w"][:LATENT, :], _vec(blk_e["layers"][0]["b"]),
            blk_e["layers"][1]["w"], _vec(blk_e["layers"][1]["b"]),
            blk_e["layers"][2]["w"], _vec(blk_e["layers"][2]["b"]),
            _vec(blk_e["ln_scale"]), _vec(blk_e["ln_offset"]), blk=1024)
        parts = _scatter_add(el, r2d, zeros_tbl)
        p0 = parts[:N_PAD]
        p1 = parts[N_PAD:]
        want_tables = i + 1 < MP_STEPS
        if want_tables:
            wsn = blocks[i + 1]["edge"]["layers"][0]["w"][LATENT:2 * LATENT, :]
            wrn = blocks[i + 1]["edge"]["layers"][0]["w"][2 * LATENT:, :]
            nl, ns, nr = _node_step_call(
                nl, p0, p1,
                blk_n["layers"][0]["w"][:LATENT, :],
                blk_n["layers"][0]["w"][LATENT:, :],
                _vec(blk_n["layers"][0]["b"]),
                blk_n["layers"][1]["w"], _vec(blk_n["layers"][1]["b"]),
                blk_n["layers"][2]["w"], _vec(blk_n["layers"][2]["b"]),
                _vec(blk_n["ln_scale"]), _vec(blk_n["ln_offset"]),
                wsn, wrn, blk=1024, want_tables=True)
        else:
            nl = _node_step_call(
                nl, p0, p1,
                blk_n["layers"][0]["w"][:LATENT, :],
                blk_n["layers"][0]["w"][LATENT:, :],
                _vec(blk_n["layers"][0]["b"]),
                blk_n["layers"][1]["w"], _vec(blk_n["layers"][1]["b"]),
                blk_n["layers"][2]["w"], _vec(blk_n["layers"][2]["b"]),
                _vec(blk_n["ln_scale"]), _vec(blk_n["ln_offset"]),
                None, None, blk=1024, want_tables=False)

    dec = params["decoder"]
    o_std = params["out_norm_std"]
    o_mean = params["out_norm_mean"]
    w3d = jnp.zeros((LATENT, LATENT), _F32).at[:, :3].set(
        dec["layers"][2]["w"] * o_std[None, :])
    b3d = jnp.zeros((LATENT,), _F32).at[:3].set(
        dec["layers"][2]["b"] * o_std + o_mean)
    acc = _decoder_call(
        nl, dec["layers"][0]["w"], _vec(dec["layers"][0]["b"]),
        dec["layers"][1]["w"], _vec(dec["layers"][1]["b"]),
        w3d, _vec(b3d), blk=1024)

    return 2 * world_pos + acc[:N_NODES, :3] - prev_world_pos
